# baseline (speedup 1.0000x reference)
import functools

import jax
import jax.numpy as jnp
from jax import lax
from jax.experimental import pallas as pl
from jax.experimental.pallas import tpu as pltpu
from jax.experimental.pallas import tpu_sc as plsc

N0 = 100000
R = 16
B = 16384
NC = 2
NSUB = 16
NW = NC * NSUB
PER_TILE = B // NW
C = 32
NCHUNK = PER_TILE // C

_mesh = plsc.VectorSubcoreMesh(
    core_axis_name="c", subcore_axis_name="s", num_cores=NC, num_subcores=NSUB
)


@functools.partial(
    pl.kernel,
    out_type=jax.ShapeDtypeStruct((B,), jnp.float32),
    mesh=_mesh,
    compiler_params=pltpu.CompilerParams(
        needs_layout_passes=False, use_tc_tiling_on_sc=True),
    scratch_types=[
        pltpu.VMEM((3 * PER_TILE,), jnp.float32),
        pltpu.VMEM((3 * PER_TILE,), jnp.float32),
        pltpu.VMEM((PER_TILE,), jnp.int32),
        pltpu.VMEM((PER_TILE,), jnp.int32),
        pltpu.VMEM((PER_TILE,), jnp.int32),
        pltpu.VMEM((PER_TILE,), jnp.int32),
        pltpu.VMEM((PER_TILE,), jnp.int32),
        pltpu.VMEM((PER_TILE,), jnp.int32),
        pltpu.VMEM((4 * PER_TILE,), jnp.int32),
        pltpu.VMEM((2 * C, 128), jnp.float32),
        pltpu.VMEM((2 * C, 128), jnp.float32),
        pltpu.VMEM((2 * C, 128), jnp.float32),
        pltpu.VMEM((2 * C, 128), jnp.float32),
        pltpu.VMEM((2 * C, 256), jnp.float32),
        pltpu.VMEM((2 * C, 256), jnp.float32),
        pltpu.VMEM((2 * C,), jnp.float32),
        pltpu.SemaphoreType.DMA,
    ],
)
def _tt_sc(xT, big, small, out, xbuf, wbuf, jlo0, jhi0, jlo1, jhi1,
           jlo2, jhi2, offbuf, rows0lo, rows0hi, rows2lo, rows2hi,
           rows1lo, rows1hi, outv, sem):
    wid = lax.axis_index("s") * NC + lax.axis_index("c")
    base0 = wid * PER_TILE

    xcp = [pltpu.async_copy(xT.at[pl.ds(i * B + base0, PER_TILE)],
                            xbuf.at[pl.ds(i * PER_TILE, PER_TILE)], sem)
           for i in range(3)]
    for cp in xcp:
        cp.wait()

    for i in range(3):
        jlo_ref = (jlo0, jlo1, jlo2)[i]
        jhi_ref = (jhi0, jhi1, jhi2)[i]
        for t in range(PER_TILE // 16):
            sl = pl.ds(t * 16, 16)
            xv = xbuf[pl.ds(i * PER_TILE + t * 16, 16)]
            xr = (xv + 1.0) * (0.5 * (N0 - 1))
            xr = jnp.minimum(jnp.maximum(xr, 0.0), float(N0 - 1))
            jlo = xr.astype(jnp.int32)
            w = xr - jlo.astype(jnp.float32)
            jhi = jnp.where(w > 0.0, jlo + 1, jlo)
            wbuf[pl.ds(i * PER_TILE + t * 16, 16)] = w
            if i == 1:
                jlo_ref[sl] = jlo
                jhi_ref[sl] = jhi
            else:
                half = 0 if i == 0 else 16
                oslot = 0 if i == 0 else 2
                jlo_ref[sl] = lax.shift_right_logical(jlo, 2)
                jhi_ref[sl] = lax.shift_right_logical(jhi, 2)
                offbuf[pl.ds(oslot * PER_TILE + t * 16, 16)] = (
                    (jlo & 3) * 32 + half)
                offbuf[pl.ds((oslot + 1) * PER_TILE + t * 16, 16)] = (
                    (jhi & 3) * 32 + half)

    def fire(k):
        ssl = pl.ds((k % 2) * C, C)
        ksl = pl.ds(k * C, C)
        return [
            pltpu.async_copy(small.at[jlo0.at[ksl]], rows0lo.at[ssl], sem),
            pltpu.async_copy(small.at[jhi0.at[ksl]], rows0hi.at[ssl], sem),
            pltpu.async_copy(small.at[jlo2.at[ksl]], rows2lo.at[ssl], sem),
            pltpu.async_copy(small.at[jhi2.at[ksl]], rows2hi.at[ssl], sem),
            pltpu.async_copy(big.at[jlo1.at[ksl]], rows1lo.at[ssl], sem),
            pltpu.async_copy(big.at[jhi1.at[ksl]], rows1hi.at[ssl], sem),
        ]

    lane = lax.iota(jnp.int32, 16)
    lane0 = lane == 0
    outcps = []
    pend = fire(0)
    for k in range(NCHUNK):
        s = k % 2
        nxt = fire(k + 1) if k + 1 < NCHUNK else []
        for cp in pend:
            cp.wait()
        pend = nxt

        def ebody(e, carry):
            ev = jnp.full((16,), e, jnp.int32) + k * C
            w0 = plsc.load_gather(wbuf, [ev])
            w1 = plsc.load_gather(wbuf, [ev + PER_TILE])
            w2 = plsc.load_gather(wbuf, [ev + 2 * PER_TILE])
            se = e + s * C
            sev = jnp.full((16,), se, jnp.int32)
            lane16 = lax.iota(jnp.int32, 16)
            o0l = plsc.load_gather(offbuf, [ev])
            o0h = plsc.load_gather(offbuf, [ev + PER_TILE])
            o2l = plsc.load_gather(offbuf, [ev + 2 * PER_TILE])
            o2h = plsc.load_gather(offbuf, [ev + 3 * PER_TILE])
            r2l = plsc.load_gather(rows2lo, [sev, lane16 + o2l])
            r2h = plsc.load_gather(rows2hi, [sev, lane16 + o2h])
            v2 = r2l + w2 * (r2h - r2l)
            r0l = plsc.load_gather(rows0lo, [sev, lane16 + o0l])
            r0h = plsc.load_gather(rows0hi, [sev, lane16 + o0h])
            v0 = r0l + w0 * (r0h - r0l)
            ul0 = jnp.zeros((R,), jnp.float32)
            ul1 = jnp.zeros((R,), jnp.float32)
            uh0 = jnp.zeros((R,), jnp.float32)
            uh1 = jnp.zeros((R,), jnp.float32)
            for a in range(R):
                v0a = v0[a]
                ml = rows1lo[se, pl.ds(a * 16, 16)]
                mh = rows1hi[se, pl.ds(a * 16, 16)]
                if a % 2 == 0:
                    ul0 = ul0 + v0a * ml
                    uh0 = uh0 + v0a * mh
                else:
                    ul1 = ul1 + v0a * ml
                    uh1 = uh1 + v0a * mh
            ul = ul0 + ul1
            uh = uh0 + uh1
            u = ul + w1 * (uh - ul)
            ans = jnp.sum(u * v2)
            plsc.store_scatter(outv, [jnp.full((16,), se, jnp.int32)],
                               jnp.full((16,), ans, jnp.float32), mask=lane0)
            return carry

        lax.fori_loop(0, C, ebody, 0)
        if len(outcps) == 2:
            outcps.pop(0).wait()
        outcps.append(pltpu.async_copy(outv.at[pl.ds(s * C, C)],
                                       out.at[pl.ds(base0 + k * C, C)], sem))
    for cp in outcps:
        cp.wait()


def kernel(x, core0, core1, core2):
    xT = x.T.reshape(3 * B)
    big = core1.transpose(1, 0, 2).reshape(N0, 2 * 128)
    c0r = core0.reshape(N0, R)
    c2r = core2.reshape(R, N0).T
    small = jnp.concatenate([c0r, c2r], axis=1).reshape(N0 // 4, 128)
    return _tt_sc(xT, big, small)


# speedup vs baseline: 1.0541x; 1.0541x over previous
import functools

import jax
import jax.numpy as jnp
from jax import lax
from jax.experimental import pallas as pl
from jax.experimental.pallas import tpu as pltpu
from jax.experimental.pallas import tpu_sc as plsc

N0 = 100000
R = 16
B = 16384
NC = 2
NSUB = 16
NW = NC * NSUB
PER_TILE = B // NW
C = 32
NCHUNK = PER_TILE // C

_mesh = plsc.VectorSubcoreMesh(
    core_axis_name="c", subcore_axis_name="s", num_cores=NC, num_subcores=NSUB
)


@functools.partial(
    pl.kernel,
    out_type=jax.ShapeDtypeStruct((B,), jnp.float32),
    mesh=_mesh,
    compiler_params=pltpu.CompilerParams(
        needs_layout_passes=False, use_tc_tiling_on_sc=True),
    scratch_types=[
        pltpu.VMEM((3 * PER_TILE,), jnp.float32),
        pltpu.VMEM((3 * PER_TILE,), jnp.float32),
        pltpu.VMEM((PER_TILE,), jnp.int32),
        pltpu.VMEM((PER_TILE,), jnp.int32),
        pltpu.VMEM((PER_TILE,), jnp.int32),
        pltpu.VMEM((PER_TILE,), jnp.int32),
        pltpu.VMEM((PER_TILE,), jnp.int32),
        pltpu.VMEM((PER_TILE,), jnp.int32),
        pltpu.VMEM((2 * C, 128), jnp.float32),
        pltpu.VMEM((2 * C, 128), jnp.float32),
        pltpu.VMEM((2 * C, 128), jnp.float32),
        pltpu.VMEM((2 * C, 128), jnp.float32),
        pltpu.VMEM((2 * C, 256), jnp.float32),
        pltpu.VMEM((2 * C, 256), jnp.float32),
        pltpu.VMEM((2 * C,), jnp.float32),
        pltpu.SemaphoreType.DMA,
    ],
)
def _tt_sc(xT, big, small, out, xbuf, wbuf, jlo0, jhi0, jlo1, jhi1,
           jlo2, jhi2, rows0lo, rows0hi, rows2lo, rows2hi,
           rows1lo, rows1hi, outv, sem):
    wid = lax.axis_index("s") * NC + lax.axis_index("c")
    base0 = wid * PER_TILE

    xcp = [pltpu.async_copy(xT.at[pl.ds(i * B + base0, PER_TILE)],
                            xbuf.at[pl.ds(i * PER_TILE, PER_TILE)], sem)
           for i in range(3)]
    for cp in xcp:
        cp.wait()

    for i in range(3):
        jlo_ref = (jlo0, jlo1, jlo2)[i]
        jhi_ref = (jhi0, jhi1, jhi2)[i]
        for t in range(PER_TILE // 16):
            sl = pl.ds(t * 16, 16)
            xv = xbuf[pl.ds(i * PER_TILE + t * 16, 16)]
            xr = (xv + 1.0) * (0.5 * (N0 - 1))
            xr = jnp.minimum(jnp.maximum(xr, 0.0), float(N0 - 1))
            jlo = xr.astype(jnp.int32)
            w = xr - jlo.astype(jnp.float32)
            jhi = jnp.where(w > 0.0, jlo + 1, jlo)
            wbuf[pl.ds(i * PER_TILE + t * 16, 16)] = w
            jlo_ref[sl] = jlo
            jhi_ref[sl] = jhi

    def fire(k):
        ssl = pl.ds((k % 2) * C, C)
        ksl = pl.ds(k * C, C)
        return [
            pltpu.async_copy(small.at[jlo0.at[ksl]], rows0lo.at[ssl], sem),
            pltpu.async_copy(small.at[jhi0.at[ksl]], rows0hi.at[ssl], sem),
            pltpu.async_copy(small.at[jlo2.at[ksl]], rows2lo.at[ssl], sem),
            pltpu.async_copy(small.at[jhi2.at[ksl]], rows2hi.at[ssl], sem),
            pltpu.async_copy(big.at[jlo1.at[ksl]], rows1lo.at[ssl], sem),
            pltpu.async_copy(big.at[jhi1.at[ksl]], rows1hi.at[ssl], sem),
        ]

    lane = lax.iota(jnp.int32, 16)
    lane0 = lane == 0
    outcps = []
    pend = fire(0)
    for k in range(NCHUNK):
        s = k % 2
        nxt = fire(k + 1) if k + 1 < NCHUNK else []
        for cp in pend:
            cp.wait()
        pend = nxt

        def ebody(e, carry):
            ev = jnp.full((16,), e, jnp.int32) + k * C
            w0 = plsc.load_gather(wbuf, [ev])
            w1 = plsc.load_gather(wbuf, [ev + PER_TILE])
            w2 = plsc.load_gather(wbuf, [ev + 2 * PER_TILE])
            se = e + s * C
            r2l = rows2lo[se, pl.ds(16, 16)]
            r2h = rows2hi[se, pl.ds(16, 16)]
            v2 = r2l + w2 * (r2h - r2l)
            r0l = rows0lo[se, pl.ds(0, 16)]
            r0h = rows0hi[se, pl.ds(0, 16)]
            v0 = r0l + w0 * (r0h - r0l)
            ul0 = jnp.zeros((R,), jnp.float32)
            ul1 = jnp.zeros((R,), jnp.float32)
            uh0 = jnp.zeros((R,), jnp.float32)
            uh1 = jnp.zeros((R,), jnp.float32)
            for a in range(R):
                v0a = v0[a]
                ml = rows1lo[se, pl.ds(a * 16, 16)]
                mh = rows1hi[se, pl.ds(a * 16, 16)]
                if a % 2 == 0:
                    ul0 = ul0 + v0a * ml
                    uh0 = uh0 + v0a * mh
                else:
                    ul1 = ul1 + v0a * ml
                    uh1 = uh1 + v0a * mh
            ul = ul0 + ul1
            uh = uh0 + uh1
            u = ul + w1 * (uh - ul)
            ans = jnp.sum(u * v2)
            plsc.store_scatter(outv, [jnp.full((16,), se, jnp.int32)],
                               jnp.full((16,), ans, jnp.float32), mask=lane0)
            return carry

        lax.fori_loop(0, C, ebody, 0)
        if len(outcps) == 2:
            outcps.pop(0).wait()
        outcps.append(pltpu.async_copy(outv.at[pl.ds(s * C, C)],
                                       out.at[pl.ds(base0 + k * C, C)], sem))
    for cp in outcps:
        cp.wait()


def kernel(x, core0, core1, core2):
    xT = x.T.reshape(3 * B)
    big = core1.transpose(1, 0, 2).reshape(N0, 2 * 128)
    c0r = core0.reshape(N0, R)
    c2r = core2.reshape(R, N0).T
    small = jnp.concatenate(
        [c0r, c2r, jnp.zeros((N0, 96), jnp.float32)], axis=1)
    return _tt_sc(xT, big, small)


# speedup vs baseline: 1.0548x; 1.0007x over previous
import functools

import jax
import jax.numpy as jnp
from jax import lax
from jax.experimental import pallas as pl
from jax.experimental.pallas import tpu as pltpu
from jax.experimental.pallas import tpu_sc as plsc

N0 = 100000
R = 16
B = 16384
NC = 2
NSUB = 16
NW = NC * NSUB
PER_TILE = B // NW
C = 32
NCHUNK = PER_TILE // C

_mesh = plsc.VectorSubcoreMesh(
    core_axis_name="c", subcore_axis_name="s", num_cores=NC, num_subcores=NSUB
)


JCH = 782


@functools.partial(
    pl.kernel,
    out_type=jax.ShapeDtypeStruct((N0, 128), jnp.float32),
    mesh=_mesh,
    compiler_params=pltpu.CompilerParams(
        needs_layout_passes=False, use_tc_tiling_on_sc=True,
        disable_bounds_checks=True),
    scratch_types=[
        pltpu.VMEM((16, 128), jnp.float32),
        pltpu.VMEM((16, 128), jnp.float32),
        pltpu.VMEM((128, 128), jnp.float32),
        pltpu.SemaphoreType.DMA,
    ],
)
def _small_build(t0, t2, out, b0, b2, orows, sem):
    wid = lax.axis_index("s") * NC + lax.axis_index("c")
    lane16 = lax.iota(jnp.int32, 16)

    def chunk(it, carry):
        ch = it * NW + wid
        jb = ch * 128

        @pl.when(ch < JCH - 1)
        def _full():
            cp0 = pltpu.async_copy(t0.at[pl.ds(0, 16), pl.ds(jb, 128)], b0, sem)
            cp1 = pltpu.async_copy(t2.at[pl.ds(0, 16), pl.ds(jb, 128)], b2, sem)
            cp0.wait()
            cp1.wait()

            def jrow(jl, c):
                jv = jnp.full((16,), jl, jnp.int32)
                orows[jl, pl.ds(0, 16)] = plsc.load_gather(b0, [lane16, jv])
                orows[jl, pl.ds(16, 16)] = plsc.load_gather(b2, [lane16, jv])
                return c

            lax.fori_loop(0, 128, jrow, 0)
            pltpu.async_copy(orows, out.at[pl.ds(jb, 128)], sem).wait()

        @pl.when(ch == JCH - 1)
        def _short():
            cp0 = pltpu.async_copy(t0.at[pl.ds(0, 16), pl.ds(jb, 128)], b0, sem)
            cp1 = pltpu.async_copy(t2.at[pl.ds(0, 16), pl.ds(jb, 128)], b2, sem)
            cp0.wait()
            cp1.wait()

            def jrow(jl, c):
                jv = jnp.full((16,), jl, jnp.int32)
                orows[jl, pl.ds(0, 16)] = plsc.load_gather(b0, [lane16, jv])
                orows[jl, pl.ds(16, 16)] = plsc.load_gather(b2, [lane16, jv])
                return c

            lax.fori_loop(0, 32, jrow, 0)
            pltpu.async_copy(orows.at[pl.ds(0, 32)],
                             out.at[pl.ds(jb, 32)], sem).wait()

        return carry

    lax.fori_loop(0, (JCH + NW - 1) // NW, chunk, 0)


@functools.partial(
    pl.kernel,
    out_type=jax.ShapeDtypeStruct((B,), jnp.float32),
    mesh=_mesh,
    compiler_params=pltpu.CompilerParams(
        needs_layout_passes=False, use_tc_tiling_on_sc=True),
    scratch_types=[
        pltpu.VMEM((3 * PER_TILE,), jnp.float32),
        pltpu.VMEM((3 * PER_TILE,), jnp.float32),
        pltpu.VMEM((PER_TILE,), jnp.int32),
        pltpu.VMEM((PER_TILE,), jnp.int32),
        pltpu.VMEM((PER_TILE,), jnp.int32),
        pltpu.VMEM((PER_TILE,), jnp.int32),
        pltpu.VMEM((PER_TILE,), jnp.int32),
        pltpu.VMEM((PER_TILE,), jnp.int32),
        pltpu.VMEM((2 * C, 128), jnp.float32),
        pltpu.VMEM((2 * C, 128), jnp.float32),
        pltpu.VMEM((2 * C, 128), jnp.float32),
        pltpu.VMEM((2 * C, 128), jnp.float32),
        pltpu.VMEM((2 * C, 256), jnp.float32),
        pltpu.VMEM((2 * C, 256), jnp.float32),
        pltpu.VMEM((2 * C,), jnp.float32),
        pltpu.SemaphoreType.DMA,
    ],
)
def _tt_sc(xT, big, small, out, xbuf, wbuf, jlo0, jhi0, jlo1, jhi1,
           jlo2, jhi2, rows0lo, rows0hi, rows2lo, rows2hi,
           rows1lo, rows1hi, outv, sem):
    wid = lax.axis_index("s") * NC + lax.axis_index("c")
    base0 = wid * PER_TILE

    xcp = [pltpu.async_copy(xT.at[pl.ds(i * B + base0, PER_TILE)],
                            xbuf.at[pl.ds(i * PER_TILE, PER_TILE)], sem)
           for i in range(3)]
    for cp in xcp:
        cp.wait()

    for i in range(3):
        jlo_ref = (jlo0, jlo1, jlo2)[i]
        jhi_ref = (jhi0, jhi1, jhi2)[i]
        for t in range(PER_TILE // 16):
            sl = pl.ds(t * 16, 16)
            xv = xbuf[pl.ds(i * PER_TILE + t * 16, 16)]
            xr = (xv + 1.0) * (0.5 * (N0 - 1))
            xr = jnp.minimum(jnp.maximum(xr, 0.0), float(N0 - 1))
            jlo = xr.astype(jnp.int32)
            w = xr - jlo.astype(jnp.float32)
            jhi = jnp.where(w > 0.0, jlo + 1, jlo)
            wbuf[pl.ds(i * PER_TILE + t * 16, 16)] = w
            jlo_ref[sl] = jlo
            jhi_ref[sl] = jhi

    def fire(k):
        ssl = pl.ds((k % 2) * C, C)
        ksl = pl.ds(k * C, C)
        return [
            pltpu.async_copy(small.at[jlo0.at[ksl]], rows0lo.at[ssl], sem),
            pltpu.async_copy(small.at[jhi0.at[ksl]], rows0hi.at[ssl], sem),
            pltpu.async_copy(small.at[jlo2.at[ksl]], rows2lo.at[ssl], sem),
            pltpu.async_copy(small.at[jhi2.at[ksl]], rows2hi.at[ssl], sem),
            pltpu.async_copy(big.at[jlo1.at[ksl]], rows1lo.at[ssl], sem),
            pltpu.async_copy(big.at[jhi1.at[ksl]], rows1hi.at[ssl], sem),
        ]

    lane = lax.iota(jnp.int32, 16)
    lane0 = lane == 0
    outcps = []
    pend = fire(0)
    for k in range(NCHUNK):
        s = k % 2
        nxt = fire(k + 1) if k + 1 < NCHUNK else []
        for cp in pend:
            cp.wait()
        pend = nxt

        def ebody(e, carry):
            ev = jnp.full((16,), e, jnp.int32) + k * C
            w0 = plsc.load_gather(wbuf, [ev])
            w1 = plsc.load_gather(wbuf, [ev + PER_TILE])
            w2 = plsc.load_gather(wbuf, [ev + 2 * PER_TILE])
            se = e + s * C
            r2l = rows2lo[se, pl.ds(16, 16)]
            r2h = rows2hi[se, pl.ds(16, 16)]
            v2 = r2l + w2 * (r2h - r2l)
            r0l = rows0lo[se, pl.ds(0, 16)]
            r0h = rows0hi[se, pl.ds(0, 16)]
            v0 = r0l + w0 * (r0h - r0l)
            ul0 = jnp.zeros((R,), jnp.float32)
            ul1 = jnp.zeros((R,), jnp.float32)
            uh0 = jnp.zeros((R,), jnp.float32)
            uh1 = jnp.zeros((R,), jnp.float32)
            for a in range(R):
                v0a = v0[a]
                ml = rows1lo[se, pl.ds(a * 16, 16)]
                mh = rows1hi[se, pl.ds(a * 16, 16)]
                if a % 2 == 0:
                    ul0 = ul0 + v0a * ml
                    uh0 = uh0 + v0a * mh
                else:
                    ul1 = ul1 + v0a * ml
                    uh1 = uh1 + v0a * mh
            ul = ul0 + ul1
            uh = uh0 + uh1
            u = ul + w1 * (uh - ul)
            ans = jnp.sum(u * v2)
            plsc.store_scatter(outv, [jnp.full((16,), se, jnp.int32)],
                               jnp.full((16,), ans, jnp.float32), mask=lane0)
            return carry

        lax.fori_loop(0, C, ebody, 0)
        if len(outcps) == 2:
            outcps.pop(0).wait()
        outcps.append(pltpu.async_copy(outv.at[pl.ds(s * C, C)],
                                       out.at[pl.ds(base0 + k * C, C)], sem))
    for cp in outcps:
        cp.wait()


def kernel(x, core0, core1, core2):
    xT = x.T.reshape(3 * B)
    big = core1.transpose(1, 0, 2).reshape(N0, 2 * 128)
    t0 = core0.transpose(0, 2, 1).reshape(R, N0)
    t2 = core2.transpose(0, 2, 1).reshape(R, N0)
    small = _small_build(t0, t2)
    return _tt_sc(xT, big, small)


# speedup vs baseline: 1.2346x; 1.1705x over previous
import functools

import jax
import jax.numpy as jnp
from jax import lax
from jax.experimental import pallas as pl
from jax.experimental.pallas import tpu as pltpu
from jax.experimental.pallas import tpu_sc as plsc

N0 = 100000
R = 16
B = 16384
NC = 2
NSUB = 16
NW = NC * NSUB
PER_TILE = B // NW
C = 32
NCHUNK = PER_TILE // C

_mesh = plsc.VectorSubcoreMesh(
    core_axis_name="c", subcore_axis_name="s", num_cores=NC, num_subcores=NSUB
)


JCH = 782


@functools.partial(
    pl.kernel,
    out_type=jax.ShapeDtypeStruct((N0, 128), jnp.float32),
    mesh=_mesh,
    compiler_params=pltpu.CompilerParams(
        needs_layout_passes=False, use_tc_tiling_on_sc=True,
        disable_bounds_checks=True),
    scratch_types=[
        pltpu.VMEM((2 * 16, 128), jnp.float32),
        pltpu.VMEM((2 * 16, 128), jnp.float32),
        pltpu.VMEM((2 * 128, 128), jnp.float32),
        pltpu.SemaphoreType.DMA,
    ],
)
def _small_build(t0, t2, out, b0, b2, orows, sem):
    wid = lax.axis_index("s") * NC + lax.axis_index("c")
    lane16 = lax.iota(jnp.int32, 16)
    NIT = (JCH + NW - 1) // NW

    def ch_of(it):
        return jnp.minimum(it * NW + wid, JCH - 1)

    def fire(it):
        s = it % 2
        jb = ch_of(it) * 128
        return [
            pltpu.async_copy(t0.at[pl.ds(0, 16), pl.ds(jb, 128)],
                             b0.at[pl.ds(s * 16, 16)], sem),
            pltpu.async_copy(t2.at[pl.ds(0, 16), pl.ds(jb, 128)],
                             b2.at[pl.ds(s * 16, 16)], sem),
        ]

    pend = fire(0)
    outcps = []
    for it in range(NIT):
        s = it % 2
        nxt = fire(it + 1) if it + 1 < NIT else []
        for cp in pend:
            cp.wait()
        pend = nxt
        ch = ch_of(it)
        jcnt = jnp.where(ch == JCH - 1, 32, 128)

        def jrow(jl, c):
            jv = jnp.full((16,), jl + s * 16, jnp.int32) * 0 + jl
            jv = jnp.full((16,), jl, jnp.int32)
            orows[jl + s * 128, pl.ds(0, 16)] = plsc.load_gather(
                b0, [lane16 + s * 16, jv])
            orows[jl + s * 128, pl.ds(16, 16)] = plsc.load_gather(
                b2, [lane16 + s * 16, jv])
            return c

        lax.fori_loop(0, jcnt, jrow, 0)
        if len(outcps) == 2:
            outcps.pop(0).wait()
        outcps.append(pltpu.async_copy(
            orows.at[pl.ds(s * 128, 128)].at[pl.ds(0, jcnt)],
            out.at[pl.ds(ch * 128, jcnt)], sem))
    for cp in outcps:
        cp.wait()


@functools.partial(
    pl.kernel,
    out_type=jax.ShapeDtypeStruct((B,), jnp.float32),
    mesh=_mesh,
    compiler_params=pltpu.CompilerParams(
        needs_layout_passes=False, use_tc_tiling_on_sc=True),
    scratch_types=[
        pltpu.VMEM((3 * PER_TILE,), jnp.float32),
        pltpu.VMEM((3 * PER_TILE,), jnp.float32),
        pltpu.VMEM((PER_TILE,), jnp.int32),
        pltpu.VMEM((PER_TILE,), jnp.int32),
        pltpu.VMEM((PER_TILE,), jnp.int32),
        pltpu.VMEM((PER_TILE,), jnp.int32),
        pltpu.VMEM((PER_TILE,), jnp.int32),
        pltpu.VMEM((PER_TILE,), jnp.int32),
        pltpu.VMEM((2 * C, 128), jnp.float32),
        pltpu.VMEM((2 * C, 128), jnp.float32),
        pltpu.VMEM((2 * C, 128), jnp.float32),
        pltpu.VMEM((2 * C, 128), jnp.float32),
        pltpu.VMEM((2 * C, 256), jnp.float32),
        pltpu.VMEM((2 * C, 256), jnp.float32),
        pltpu.VMEM((2 * C,), jnp.float32),
        pltpu.SemaphoreType.DMA,
    ],
)
def _tt_sc(xT, big, small, out, xbuf, wbuf, jlo0, jhi0, jlo1, jhi1,
           jlo2, jhi2, rows0lo, rows0hi, rows2lo, rows2hi,
           rows1lo, rows1hi, outv, sem):
    wid = lax.axis_index("s") * NC + lax.axis_index("c")
    base0 = wid * PER_TILE

    xcp = [pltpu.async_copy(xT.at[pl.ds(i * B + base0, PER_TILE)],
                            xbuf.at[pl.ds(i * PER_TILE, PER_TILE)], sem)
           for i in range(3)]
    for cp in xcp:
        cp.wait()

    for i in range(3):
        jlo_ref = (jlo0, jlo1, jlo2)[i]
        jhi_ref = (jhi0, jhi1, jhi2)[i]
        for t in range(PER_TILE // 16):
            sl = pl.ds(t * 16, 16)
            xv = xbuf[pl.ds(i * PER_TILE + t * 16, 16)]
            xr = (xv + 1.0) * (0.5 * (N0 - 1))
            xr = jnp.minimum(jnp.maximum(xr, 0.0), float(N0 - 1))
            jlo = xr.astype(jnp.int32)
            w = xr - jlo.astype(jnp.float32)
            jhi = jnp.where(w > 0.0, jlo + 1, jlo)
            wbuf[pl.ds(i * PER_TILE + t * 16, 16)] = w
            jlo_ref[sl] = jlo
            jhi_ref[sl] = jhi

    def fire(k):
        ssl = pl.ds((k % 2) * C, C)
        ksl = pl.ds(k * C, C)
        return [
            pltpu.async_copy(small.at[jlo0.at[ksl]], rows0lo.at[ssl], sem),
            pltpu.async_copy(small.at[jhi0.at[ksl]], rows0hi.at[ssl], sem),
            pltpu.async_copy(small.at[jlo2.at[ksl]], rows2lo.at[ssl], sem),
            pltpu.async_copy(small.at[jhi2.at[ksl]], rows2hi.at[ssl], sem),
            pltpu.async_copy(big.at[jlo1.at[ksl]], rows1lo.at[ssl], sem),
            pltpu.async_copy(big.at[jhi1.at[ksl]], rows1hi.at[ssl], sem),
        ]

    lane = lax.iota(jnp.int32, 16)
    lane0 = lane == 0
    outcps = []
    pend = fire(0)
    for k in range(NCHUNK):
        s = k % 2
        nxt = fire(k + 1) if k + 1 < NCHUNK else []
        for cp in pend:
            cp.wait()
        pend = nxt

        def ebody(e, carry):
            ev = jnp.full((16,), e, jnp.int32) + k * C
            w0 = plsc.load_gather(wbuf, [ev])
            w1 = plsc.load_gather(wbuf, [ev + PER_TILE])
            w2 = plsc.load_gather(wbuf, [ev + 2 * PER_TILE])
            se = e + s * C
            r2l = rows2lo[se, pl.ds(16, 16)]
            r2h = rows2hi[se, pl.ds(16, 16)]
            v2 = r2l + w2 * (r2h - r2l)
            r0l = rows0lo[se, pl.ds(0, 16)]
            r0h = rows0hi[se, pl.ds(0, 16)]
            v0 = r0l + w0 * (r0h - r0l)
            ul0 = jnp.zeros((R,), jnp.float32)
            ul1 = jnp.zeros((R,), jnp.float32)
            uh0 = jnp.zeros((R,), jnp.float32)
            uh1 = jnp.zeros((R,), jnp.float32)
            for a in range(R):
                v0a = v0[a]
                ml = rows1lo[se, pl.ds(a * 16, 16)]
                mh = rows1hi[se, pl.ds(a * 16, 16)]
                if a % 2 == 0:
                    ul0 = ul0 + v0a * ml
                    uh0 = uh0 + v0a * mh
                else:
                    ul1 = ul1 + v0a * ml
                    uh1 = uh1 + v0a * mh
            ul = ul0 + ul1
            uh = uh0 + uh1
            u = ul + w1 * (uh - ul)
            ans = jnp.sum(u * v2)
            plsc.store_scatter(outv, [jnp.full((16,), se, jnp.int32)],
                               jnp.full((16,), ans, jnp.float32), mask=lane0)
            return carry

        lax.fori_loop(0, C, ebody, 0)
        if len(outcps) == 2:
            outcps.pop(0).wait()
        outcps.append(pltpu.async_copy(outv.at[pl.ds(s * C, C)],
                                       out.at[pl.ds(base0 + k * C, C)], sem))
    for cp in outcps:
        cp.wait()


def kernel(x, core0, core1, core2):
    xT = x.T.reshape(3 * B)
    big = core1.transpose(1, 0, 2).reshape(N0, 2 * 128)
    t0 = core0.transpose(0, 2, 1).reshape(R, N0)
    t2 = core2.transpose(0, 2, 1).reshape(R, N0)
    small = _small_build(t0, t2)
    return _tt_sc(xT, big, small)
